# baseline (speedup 1.0000x reference)
import functools

import jax
import jax.numpy as jnp
from jax import lax
from jax.experimental import pallas as pl
from jax.experimental.pallas import tpu as pltpu
from jax.experimental.pallas import tpu_sc as plsc


def _sc_gather_lines(dt2, rd2, p_dt, p_rd):
    B = p_dt.shape[0]
    info = plsc.get_sparse_core_info()
    nw = info.num_cores * info.num_subcores
    bpw = B // nw
    ck = bpw // 2

    mesh = plsc.VectorSubcoreMesh(core_axis_name="c", subcore_axis_name="s")

    @functools.partial(
        pl.kernel,
        mesh=mesh,
        out_type=(
            jax.ShapeDtypeStruct((B, 128), jnp.float32),
            jax.ShapeDtypeStruct((B, 128), jnp.float32),
        ),
        scratch_types=[
            pltpu.VMEM((bpw,), jnp.int32),
            pltpu.VMEM((bpw,), jnp.int32),
            pltpu.VMEM((ck, 128), jnp.float32),
            pltpu.VMEM((ck, 128), jnp.float32),
            pltpu.SemaphoreType.DMA,
            pltpu.SemaphoreType.DMA,
        ],
    )
    def gather_k(dt_hbm, rd_hbm, pdt_hbm, prd_hbm, out_dt, out_rd,
                 pdt_v, prd_v, dt_buf, rd_buf, sem_dt, sem_rd):
        wid = lax.axis_index("s") * info.num_cores + lax.axis_index("c")
        base = wid * bpw
        pltpu.sync_copy(pdt_hbm.at[pl.ds(base, bpw)], pdt_v)
        pltpu.sync_copy(prd_hbm.at[pl.ds(base, bpw)], prd_v)
        for half in range(2):
            cp_dt = pltpu.async_copy(
                dt_hbm.at[pdt_v.at[pl.ds(half * ck, ck)]], dt_buf, sem_dt)
            cp_rd = pltpu.async_copy(
                rd_hbm.at[prd_v.at[pl.ds(half * ck, ck)]], rd_buf, sem_rd)
            cp_dt.wait()
            cp_rd.wait()
            off = base + half * ck
            pltpu.sync_copy(dt_buf, out_dt.at[pl.ds(off, ck)])
            pltpu.sync_copy(rd_buf, out_rd.at[pl.ds(off, ck)])

    return gather_k(dt2, rd2, p_dt, p_rd)


def _tc_select_mlp(lines_dt, lines_rd, sub_dt, sub_rd, w1, w2, b2d):
    B = lines_dt.shape[0]
    d_dt = w1.shape[0]
    d_rd = w2.shape[0]
    hid = w1.shape[1]
    blk = 2048
    grid = (B // blk,)

    def body(ldt_ref, lrd_ref, sdt_ref, srd_ref, w1_ref, w2_ref, b_ref,
             o_ref):
        sdt = sdt_ref[...]
        srd = srd_ref[...]
        row_dt = jnp.zeros((blk, d_dt), jnp.float32)
        row_rd = jnp.zeros((blk, d_rd), jnp.float32)
        for s in range(128 // d_dt):
            row_dt += (ldt_ref[:, s * d_dt:(s + 1) * d_dt]
                       * (sdt == s).astype(jnp.float32))
        for s in range(128 // d_rd):
            row_rd += (lrd_ref[:, s * d_rd:(s + 1) * d_rd]
                       * (srd == s).astype(jnp.float32))
        acc = jnp.dot(row_dt, w1_ref[...], preferred_element_type=jnp.float32)
        acc += jnp.dot(row_rd, w2_ref[...], preferred_element_type=jnp.float32)
        o_ref[...] = jnp.maximum(acc + b_ref[...], 0.0)

    return pl.pallas_call(
        body,
        grid=grid,
        in_specs=[
            pl.BlockSpec((blk, 128), lambda i: (i, 0)),
            pl.BlockSpec((blk, 128), lambda i: (i, 0)),
            pl.BlockSpec((blk, 1), lambda i: (i, 0)),
            pl.BlockSpec((blk, 1), lambda i: (i, 0)),
            pl.BlockSpec(w1.shape, lambda i: (0, 0)),
            pl.BlockSpec(w2.shape, lambda i: (0, 0)),
            pl.BlockSpec(b2d.shape, lambda i: (0, 0)),
        ],
        out_specs=pl.BlockSpec((blk, hid), lambda i: (i, 0)),
        out_shape=jax.ShapeDtypeStruct((B, hid), jnp.float32),
    )(lines_dt, lines_rd, sub_dt, sub_rd, w1, w2, b2d)


def kernel(x, dt_table, rd_table, W, b):
    d_dt = dt_table.shape[1]
    d_rd = rd_table.shape[1]
    r_dt = 128 // d_dt
    r_rd = 128 // d_rd
    idx_dt = x[:, 0]
    idx_rd = x[:, 1]
    dt2 = dt_table.reshape(dt_table.shape[0] // r_dt, 128)
    rd2 = rd_table.reshape(rd_table.shape[0] // r_rd, 128)
    lines_dt, lines_rd = _sc_gather_lines(
        dt2, rd2, idx_dt // r_dt, idx_rd // r_rd)
    sub_dt = (idx_dt % r_dt).reshape(-1, 1)
    sub_rd = (idx_rd % r_rd).reshape(-1, 1)
    w1 = W[:d_dt]
    w2 = W[d_dt:]
    return _tc_select_mlp(lines_dt, lines_rd, sub_dt, sub_rd, w1, w2,
                          b.reshape(1, -1))
